# Initial kernel scaffold; baseline (speedup 1.0000x reference)
import jax
import jax.numpy as jnp
from jax.experimental import pallas as pl


def kernel(x, table):
    raise NotImplementedError("write your pallas kernel here")


# speedup vs baseline: 2.9708x; 2.9708x over previous
import functools

import jax
import jax.numpy as jnp
from jax import lax
from jax.experimental import pallas as pl
from jax.experimental.pallas import tpu as pltpu
from jax.experimental.pallas import tpu_sc as plsc

D = 128
NC, NS = 2, 16
NW = NC * NS
CH = 128
NCH = 50


def _make_gather(nch):
    mesh = plsc.VectorSubcoreMesh(core_axis_name="c", subcore_axis_name="s")

    @functools.partial(
        pl.kernel,
        out_type=jax.ShapeDtypeStruct((NW, nch, CH, D), jnp.float32),
        mesh=mesh,
        scratch_types=[
            pltpu.VMEM((nch, CH), jnp.int32),
            pltpu.VMEM((CH, D), jnp.float32),
            pltpu.SemaphoreType.DMA,
        ],
    )
    def _gather(idx_hbm, table_hbm, out_hbm, idx_v, rows_v, sem):
        wid = lax.axis_index("s") * NC + lax.axis_index("c")
        pltpu.sync_copy(idx_hbm.at[wid], idx_v)

        def step(j, carry):
            pltpu.async_copy(table_hbm.at[idx_v.at[j]], rows_v, sem).wait()
            pltpu.sync_copy(rows_v, out_hbm.at[wid, j])
            return carry

        lax.fori_loop(0, nch, step, 0)

    return _gather


_GATHER = _make_gather(NCH)


def kernel(x, table):
    b, h = x.shape
    total = b * h
    assert total == NW * NCH * CH
    idx = x.reshape(NW, NCH, CH).astype(jnp.int32)
    out = _GATHER(idx, table)
    return out.reshape(b, h, D)


# speedup vs baseline: 3.3383x; 1.1237x over previous
import functools

import jax
import jax.numpy as jnp
from jax import lax
from jax.experimental import pallas as pl
from jax.experimental.pallas import tpu as pltpu
from jax.experimental.pallas import tpu_sc as plsc

D = 128
NC, NS = 2, 16
NW = NC * NS
CH = 128
NCH = 50
NBUF = 5


def _make_gather(nch):
    mesh = plsc.VectorSubcoreMesh(core_axis_name="c", subcore_axis_name="s")

    @functools.partial(
        pl.kernel,
        out_type=jax.ShapeDtypeStruct((NW, nch, CH, D), jnp.float32),
        mesh=mesh,
        scratch_types=[
            pltpu.VMEM((nch, CH), jnp.int32),
            pltpu.VMEM((NBUF, CH, D), jnp.float32),
        ]
        + [pltpu.SemaphoreType.DMA] * (2 * NBUF),
    )
    def _gather(idx_hbm, table_hbm, out_hbm, idx_v, rows_v, *sems):
        gsem, wsem = sems[:NBUF], sems[NBUF:]
        wid = lax.axis_index("s") * NC + lax.axis_index("c")
        pltpu.sync_copy(idx_hbm.at[wid], idx_v)

        for b in range(NBUF):
            pltpu.async_copy(table_hbm.at[idx_v.at[b]], rows_v.at[b], gsem[b])

        def outer(o, carry):
            for b in range(NBUF):
                j = o * NBUF + b
                bp = (b - 1) % NBUF
                jp = j - 1
                jpc = jnp.maximum(jp, 0)

                @pl.when(jp >= 0)
                def _drain_prev_write():
                    pltpu.make_async_copy(
                        rows_v.at[bp], out_hbm.at[wid, jpc], wsem[bp]
                    ).wait()

                @pl.when((jp >= 0) & (jp + NBUF < nch))
                def _refill_prev_buf():
                    pltpu.async_copy(
                        table_hbm.at[idx_v.at[jpc + NBUF]], rows_v.at[bp], gsem[bp]
                    )

                pltpu.make_async_copy(
                    table_hbm.at[idx_v.at[j]], rows_v.at[b], gsem[b]
                ).wait()
                pltpu.async_copy(rows_v.at[b], out_hbm.at[wid, j], wsem[b])
            return carry

        lax.fori_loop(0, nch // NBUF, outer, 0)
        bl = (nch - 1) % NBUF
        pltpu.make_async_copy(
            rows_v.at[bl], out_hbm.at[wid, nch - 1], wsem[bl]
        ).wait()

    return _gather


_GATHER = _make_gather(NCH)


def kernel(x, table):
    b, h = x.shape
    total = b * h
    assert total == NW * NCH * CH
    idx = x.reshape(NW, NCH, CH).astype(jnp.int32)
    out = _GATHER(idx, table)
    return out.reshape(b, h, D)


# speedup vs baseline: 5.9083x; 1.7698x over previous
import functools

import jax
import jax.numpy as jnp
from jax import lax
from jax.experimental import pallas as pl
from jax.experimental.pallas import tpu as pltpu
from jax.experimental.pallas import tpu_sc as plsc

D = 128
NC, NS = 2, 16
NW = NC * NS
H = 50
HP = 56
BPW = 128
NBUF = 4


def _make_gather(nb, nbpw):
    mesh = plsc.VectorSubcoreMesh(core_axis_name="c", subcore_axis_name="s")

    @functools.partial(
        pl.kernel,
        out_type=jax.ShapeDtypeStruct((nb, H, D), jnp.float32),
        mesh=mesh,
        scratch_types=[
            pltpu.VMEM((nbpw * HP,), jnp.int32),
            pltpu.VMEM((NBUF, H, D), jnp.float32),
        ]
        + [pltpu.SemaphoreType.DMA] * (2 * NBUF),
    )
    def _gather(idx_hbm, table_hbm, out_hbm, idx_v, rows_v, *sems):
        gsem, wsem = sems[:NBUF], sems[NBUF:]
        wid = lax.axis_index("s") * NC + lax.axis_index("c")
        pltpu.sync_copy(idx_hbm.at[pl.ds(wid * nbpw * HP, nbpw * HP)], idx_v)

        def fire_gather(s, b):
            pltpu.async_copy(
                table_hbm.at[idx_v.at[pl.ds(s * HP, H)]], rows_v.at[b], gsem[b]
            )

        def drain_gather(b):
            pltpu.make_async_copy(
                table_hbm.at[idx_v.at[pl.ds(0, H)]], rows_v.at[b], gsem[b]
            ).wait()

        def fire_write(s, b):
            pltpu.async_copy(rows_v.at[b], out_hbm.at[wid * nbpw + s], wsem[b])

        def drain_write(b):
            pltpu.make_async_copy(rows_v.at[b], out_hbm.at[0], wsem[b]).wait()

        for b in range(NBUF):
            fire_gather(b, b)

        def outer(o, carry):
            for bi in range(NBUF):
                s = o * NBUF + bi
                bp = (bi - 1) % NBUF
                sp = s - 1
                spc = jnp.maximum(sp, 0)

                @pl.when(sp >= 0)
                def _drain_prev_write():
                    drain_write(bp)

                @pl.when((sp >= 0) & (sp + NBUF < nbpw))
                def _refill_prev_buf():
                    fire_gather(spc + NBUF, bp)

                drain_gather(bi)
                fire_write(s, bi)
            return carry

        lax.fori_loop(0, nbpw // NBUF, outer, 0)
        drain_write((nbpw - 1) % NBUF)

    return _gather


_GATHER = _make_gather(4096, BPW)


def kernel(x, table):
    b, h = x.shape
    assert h == H and b == NW * BPW
    idx = jnp.pad(x.astype(jnp.int32), ((0, 0), (0, HP - H))).reshape(-1)
    return _GATHER(idx, table)


# speedup vs baseline: 10.7038x; 1.8116x over previous
import functools

import jax
import jax.numpy as jnp
from jax import lax
from jax.experimental import pallas as pl
from jax.experimental.pallas import tpu as pltpu
from jax.experimental.pallas import tpu_sc as plsc

D = 128
NC, NS = 2, 16
NW = NC * NS
CH = 128
NCH = 50
NBUF = 5


def _make_gather(nch, nb):
    mesh = plsc.VectorSubcoreMesh(core_axis_name="c", subcore_axis_name="s")

    @functools.partial(
        pl.kernel,
        out_type=jax.ShapeDtypeStruct((nch, nb, D), jnp.float32),
        mesh=mesh,
        scratch_types=[
            pltpu.VMEM((nch, CH), jnp.int32),
            pltpu.VMEM((NBUF, CH, D), jnp.float32),
        ]
        + [pltpu.SemaphoreType.DMA] * (2 * NBUF),
    )
    def _gather(idx_hbm, table_hbm, out_hbm, idx_v, rows_v, *sems):
        gsem, wsem = sems[:NBUF], sems[NBUF:]
        wid = lax.axis_index("s") * NC + lax.axis_index("c")
        pltpu.sync_copy(idx_hbm.at[:, wid], idx_v)

        def fire_gather(s, b):
            pltpu.async_copy(table_hbm.at[idx_v.at[s]], rows_v.at[b], gsem[b])

        def drain_gather(b):
            pltpu.make_async_copy(
                table_hbm.at[idx_v.at[0]], rows_v.at[b], gsem[b]
            ).wait()

        def fire_write(s, b):
            pltpu.async_copy(
                rows_v.at[b], out_hbm.at[s, pl.ds(wid * CH, CH)], wsem[b]
            )

        def drain_write(b):
            pltpu.make_async_copy(
                rows_v.at[b], out_hbm.at[0, pl.ds(0, CH)], wsem[b]
            ).wait()

        for b in range(3):
            fire_gather(b, b)

        def outer(o, carry):
            for bi in range(NBUF):
                s = o * NBUF + bi
                bw = (bi - 2) % NBUF
                bg = (bi + 3) % NBUF

                @pl.when(s >= 2)
                def _drain_old_write():
                    drain_write(bw)

                @pl.when(s + 3 < nch)
                def _refill():
                    fire_gather(s + 3, bg)

                drain_gather(bi)
                fire_write(s, bi)
            return carry

        lax.fori_loop(0, nch // NBUF, outer, 0)
        drain_write((nch - 2) % NBUF)
        drain_write((nch - 1) % NBUF)

    return _gather


_GATHER = _make_gather(NCH, 4096)


def kernel(x, table):
    b, h = x.shape
    assert h == NCH and b == NW * CH
    idx = jnp.swapaxes(x, 0, 1).astype(jnp.int32).reshape(h, NW, CH)
    out = _GATHER(idx, table)
    return jnp.swapaxes(out, 0, 1)
